# baseline (speedup 1.0000x reference)
import functools

import jax
import jax.numpy as jnp
from jax import lax
from jax.experimental import pallas as pl
from jax.experimental.pallas import tpu as pltpu
from jax.experimental.pallas import tpu_sc as plsc

IN_F = 2048
OUT_F = 2048
RDIM = 32
KSEL = 8
SCALE = 64.0 / 32.0
NTOK = 8192
BL = 1024

NC = 2
NS = 16
NW = NC * NS
TOK_W = NTOK // NW
NGRP = TOK_W // 16


def _p1_kernel(x_ref, a_ref, d_ref, yt_ref, ybt_ref):
    y = jax.lax.dot_general(
        x_ref[...], a_ref[...], (((1,), (1,)), ((), ())),
        preferred_element_type=jnp.float32)
    yt = jnp.transpose(y)
    yt_ref[...] = yt
    ybt_ref[...] = jnp.abs(yt + jnp.transpose(d_ref[...]))


def _p1_call(x, A, d2):
    return pl.pallas_call(
        _p1_kernel,
        grid=(NTOK // BL,),
        in_specs=[
            pl.BlockSpec((BL, IN_F), lambda i: (i, 0)),
            pl.BlockSpec((RDIM, IN_F), lambda i: (0, 0)),
            pl.BlockSpec((1, RDIM), lambda i: (0, 0)),
        ],
        out_specs=[
            pl.BlockSpec((RDIM, BL), lambda i: (0, i)),
            pl.BlockSpec((RDIM, BL), lambda i: (0, i)),
        ],
        out_shape=[
            jax.ShapeDtypeStruct((RDIM, NTOK), jnp.float32),
            jax.ShapeDtypeStruct((RDIM, NTOK), jnp.float32),
        ],
        compiler_params=pltpu.CompilerParams(
            dimension_semantics=("parallel",)),
    )(x, A, d2)


@functools.partial(
    pl.kernel,
    out_type=jax.ShapeDtypeStruct((RDIM, NTOK), jnp.float32),
    mesh=plsc.VectorSubcoreMesh(core_axis_name="c", subcore_axis_name="s"),
    scratch_types=[
        pltpu.VMEM((RDIM, TOK_W), jnp.float32),
        pltpu.VMEM((RDIM, TOK_W), jnp.float32),
        pltpu.VMEM((RDIM, TOK_W), jnp.float32),
    ],
)
def _sc_topk(yt_hbm, ybt_hbm, act_hbm, yt_v, ybt_v, act_v):
    wid = lax.axis_index("s") * NC + lax.axis_index("c")
    base = wid * TOK_W
    pltpu.sync_copy(ybt_hbm.at[:, pl.ds(base, TOK_W)], ybt_v)
    pltpu.sync_copy(yt_hbm.at[:, pl.ds(base, TOK_W)], yt_v)

    neg = jnp.full((16,), -1.0, jnp.float32)
    zero = jnp.zeros((16,), jnp.float32)
    one = jnp.full((16,), 1.0, jnp.float32)
    scale = jnp.full((16,), SCALE, jnp.float32)

    def grp(g, carry):
        sl = pl.ds(g * 16, 16)
        w = [ybt_v[r, sl] for r in range(RDIM)]
        for _ in range(KSEL):
            m = w[0]
            for r in range(1, RDIM):
                m = jnp.maximum(m, w[r])
            taken = zero
            for r in range(RDIM):
                hit = jnp.where(w[r] == m, one, zero)
                cond = hit * (one - taken)
                taken = taken + cond
                w[r] = w[r] + cond * (neg - w[r])
        for r in range(RDIM):
            act_v[r, sl] = jnp.where(w[r] < zero, yt_v[r, sl] * scale, zero)
        return carry

    lax.fori_loop(0, NGRP, grp, 0)
    pltpu.sync_copy(act_v, act_hbm.at[:, pl.ds(base, TOK_W)])


def _p3_kernel(act_ref, b_ref, o_ref):
    act = act_ref[...].astype(jnp.bfloat16)
    out = jax.lax.dot_general(
        act, b_ref[...], (((0,), (1,)), ((), ())),
        preferred_element_type=jnp.float32)
    o_ref[...] = out


def _p3_call(act_t, b_bf):
    return pl.pallas_call(
        _p3_kernel,
        grid=(NTOK // BL,),
        in_specs=[
            pl.BlockSpec((RDIM, BL), lambda i: (0, i)),
            pl.BlockSpec((OUT_F, RDIM), lambda i: (0, 0)),
        ],
        out_specs=pl.BlockSpec((BL, OUT_F), lambda i: (i, 0)),
        out_shape=jax.ShapeDtypeStruct((NTOK, OUT_F), jnp.float32),
        compiler_params=pltpu.CompilerParams(
            dimension_semantics=("parallel",)),
    )(act_t, b_bf)


@jax.jit
def kernel(x, A, B, d):
    d2 = d.reshape(1, RDIM)
    b_bf = B.astype(jnp.bfloat16)
    yt, ybt = _p1_call(x, A, d2)
    act_t = _sc_topk(yt, ybt)
    return _p3_call(act_t, b_bf)


# speedup vs baseline: 1.1050x; 1.1050x over previous
import functools

import jax
import jax.numpy as jnp
from jax import lax
from jax.experimental import pallas as pl
from jax.experimental.pallas import tpu as pltpu
from jax.experimental.pallas import tpu_sc as plsc

IN_F = 2048
OUT_F = 2048
RDIM = 32
KSEL = 8
SCALE = 64.0 / 32.0
NTOK = 8192
BL = 1024
NCHUNK = 2
CTOK = NTOK // NCHUNK

NC = 2
NS = 16
NW = NC * NS
TOK_W = CTOK // NW
NGRP = TOK_W // 16


def _p1_kernel(x_ref, a_ref, d_ref, yt_ref, ybt_ref):
    y = jax.lax.dot_general(
        x_ref[...], a_ref[...], (((1,), (1,)), ((), ())),
        preferred_element_type=jnp.float32)
    yt = jnp.transpose(y)
    yt_ref[...] = yt
    ybt_ref[...] = jnp.abs(yt + jnp.transpose(d_ref[...]))


def _p1_call(x, A, d2, chunk):
    base = chunk * (CTOK // BL)
    return pl.pallas_call(
        _p1_kernel,
        grid=(CTOK // BL,),
        in_specs=[
            pl.BlockSpec((BL, IN_F), lambda i: (base + i, 0)),
            pl.BlockSpec((RDIM, IN_F), lambda i: (0, 0)),
            pl.BlockSpec((1, RDIM), lambda i: (0, 0)),
        ],
        out_specs=[
            pl.BlockSpec((RDIM, BL), lambda i: (0, i)),
            pl.BlockSpec((RDIM, BL), lambda i: (0, i)),
        ],
        out_shape=[
            jax.ShapeDtypeStruct((RDIM, CTOK), jnp.float32),
            jax.ShapeDtypeStruct((RDIM, CTOK), jnp.float32),
        ],
        compiler_params=pltpu.CompilerParams(
            dimension_semantics=("arbitrary",)),
    )(x, A, d2)


@functools.partial(
    pl.kernel,
    out_type=jax.ShapeDtypeStruct((RDIM, CTOK), jnp.float32),
    mesh=plsc.VectorSubcoreMesh(core_axis_name="c", subcore_axis_name="s"),
    scratch_types=[
        pltpu.VMEM((RDIM, TOK_W), jnp.float32),
        pltpu.VMEM((RDIM, TOK_W), jnp.float32),
        pltpu.VMEM((RDIM, TOK_W), jnp.float32),
    ],
)
def _sc_topk(yt_hbm, ybt_hbm, act_hbm, yt_v, ybt_v, act_v):
    wid = lax.axis_index("s") * NC + lax.axis_index("c")
    base = wid * TOK_W
    pltpu.sync_copy(ybt_hbm.at[:, pl.ds(base, TOK_W)], ybt_v)
    pltpu.sync_copy(yt_hbm.at[:, pl.ds(base, TOK_W)], yt_v)

    neg = jnp.full((16,), -1.0, jnp.float32)
    zero = jnp.zeros((16,), jnp.float32)
    one = jnp.full((16,), 1.0, jnp.float32)
    scale = jnp.full((16,), SCALE, jnp.float32)

    def grp(g, carry):
        sl = pl.ds(g * 16, 16)
        w = [ybt_v[r, sl] for r in range(RDIM)]
        for _ in range(KSEL):
            m = w[0]
            for r in range(1, RDIM):
                m = jnp.maximum(m, w[r])
            taken = zero
            for r in range(RDIM):
                hit = jnp.where(w[r] == m, one, zero)
                cond = hit * (one - taken)
                taken = taken + cond
                w[r] = w[r] + cond * (neg - w[r])
        for r in range(RDIM):
            act_v[r, sl] = jnp.where(w[r] < zero, yt_v[r, sl] * scale, zero)
        return carry

    lax.fori_loop(0, NGRP, grp, 0)
    pltpu.sync_copy(act_v, act_hbm.at[:, pl.ds(base, TOK_W)])


def _p3_first_kernel(act_ref, b_ref, o_ref):
    act = act_ref[...].astype(jnp.bfloat16)
    out = jax.lax.dot_general(
        act, b_ref[...], (((0,), (1,)), ((), ())),
        preferred_element_type=jnp.float32)
    o_ref[...] = out


def _p3_rest_kernel(o_in_ref, act_ref, b_ref, o_ref):
    del o_in_ref
    _p3_first_kernel(act_ref, b_ref, o_ref)


def _p3_call(act_t, b_bf, chunk, prev_out):
    base = chunk * (CTOK // BL)
    out_spec = pl.BlockSpec((BL, OUT_F), lambda i: (base + i, 0))
    out_shape = jax.ShapeDtypeStruct((NTOK, OUT_F), jnp.float32)
    act_spec = pl.BlockSpec((RDIM, BL), lambda i: (0, i))
    b_spec = pl.BlockSpec((OUT_F, RDIM), lambda i: (0, 0))
    params = pltpu.CompilerParams(dimension_semantics=("arbitrary",))
    if prev_out is None:
        return pl.pallas_call(
            _p3_first_kernel,
            grid=(CTOK // BL,),
            in_specs=[act_spec, b_spec],
            out_specs=out_spec,
            out_shape=out_shape,
            compiler_params=params,
        )(act_t, b_bf)
    return pl.pallas_call(
        _p3_rest_kernel,
        grid=(CTOK // BL,),
        in_specs=[
            pl.BlockSpec(memory_space=pl.ANY),
            act_spec,
            b_spec,
        ],
        out_specs=out_spec,
        out_shape=out_shape,
        input_output_aliases={0: 0},
        compiler_params=params,
    )(prev_out, act_t, b_bf)


@jax.jit
def kernel(x, A, B, d):
    d2 = d.reshape(1, RDIM)
    b_bf = B.astype(jnp.bfloat16)
    acts = []
    for c in range(NCHUNK):
        yt, ybt = _p1_call(x, A, d2, c)
        acts.append(_sc_topk(yt, ybt))
    out = None
    for c in range(NCHUNK):
        out = _p3_call(acts[c], b_bf, c, out)
    return out


# speedup vs baseline: 1.1733x; 1.0618x over previous
import functools

import jax
import jax.numpy as jnp
from jax import lax
from jax.experimental import pallas as pl
from jax.experimental.pallas import tpu as pltpu
from jax.experimental.pallas import tpu_sc as plsc

IN_F = 2048
OUT_F = 2048
RDIM = 32
KSEL = 8
SCALE = 64.0 / 32.0
NTOK = 8192
BL = 1024
NCHUNK = 2
CTOK = NTOK // NCHUNK

NC = 2
NS = 16
NW = NC * NS
TOK_W = CTOK // NW
NGRP = TOK_W // 16


def _p1_kernel(x_ref, a_ref, d_ref, yt_ref, ybt_ref):
    y = jax.lax.dot_general(
        x_ref[...], a_ref[...], (((1,), (1,)), ((), ())),
        preferred_element_type=jnp.float32)
    yt = jnp.transpose(y)
    yt_ref[...] = yt
    ybt_ref[...] = jnp.abs(yt + jnp.transpose(d_ref[...]))


def _p1_call(x, A, d2, chunk):
    base = chunk * (CTOK // BL)
    return pl.pallas_call(
        _p1_kernel,
        grid=(CTOK // BL,),
        in_specs=[
            pl.BlockSpec((BL, IN_F), lambda i: (base + i, 0)),
            pl.BlockSpec((RDIM, IN_F), lambda i: (0, 0)),
            pl.BlockSpec((1, RDIM), lambda i: (0, 0)),
        ],
        out_specs=[
            pl.BlockSpec((RDIM, BL), lambda i: (0, i)),
            pl.BlockSpec((RDIM, BL), lambda i: (0, i)),
        ],
        out_shape=[
            jax.ShapeDtypeStruct((RDIM, CTOK), jnp.float32),
            jax.ShapeDtypeStruct((RDIM, CTOK), jnp.float32),
        ],
        compiler_params=pltpu.CompilerParams(
            dimension_semantics=("arbitrary",)),
    )(x, A, d2)


@functools.partial(
    pl.kernel,
    out_type=jax.ShapeDtypeStruct((RDIM, CTOK), jnp.float32),
    mesh=plsc.VectorSubcoreMesh(core_axis_name="c", subcore_axis_name="s"),
    scratch_types=[
        pltpu.VMEM((RDIM, TOK_W), jnp.float32),
        pltpu.VMEM((RDIM, TOK_W), jnp.float32),
        pltpu.VMEM((RDIM, TOK_W), jnp.float32),
    ],
)
def _sc_topk(yt_hbm, ybt_hbm, act_hbm, yt_v, ybt_v, act_v):
    wid = lax.axis_index("s") * NC + lax.axis_index("c")
    base = wid * TOK_W
    pltpu.sync_copy(ybt_hbm.at[:, pl.ds(base, TOK_W)], ybt_v)
    pltpu.sync_copy(yt_hbm.at[:, pl.ds(base, TOK_W)], yt_v)

    neg = jnp.full((16,), -1.0, jnp.float32)
    zero = jnp.zeros((16,), jnp.float32)
    scale = jnp.full((16,), SCALE, jnp.float32)
    prio = [jnp.full((16,), float(RDIM - r), jnp.float32) for r in range(RDIM)]

    def _tree_max(vals):
        while len(vals) > 1:
            vals = [jnp.maximum(vals[i], vals[i + 1])
                    for i in range(0, len(vals) - 1, 2)] + (
                        [vals[-1]] if len(vals) % 2 else [])
        return vals[0]

    def one_group(sl):
        w = [ybt_v[r, sl] for r in range(RDIM)]
        for _ in range(KSEL):
            m = _tree_max(w)
            score = [jnp.where(w[r] == m, prio[r], zero) for r in range(RDIM)]
            best = _tree_max(score)
            w = [jnp.where(score[r] == best, neg, w[r]) for r in range(RDIM)]
        for r in range(RDIM):
            act_v[r, sl] = jnp.where(w[r] < zero, yt_v[r, sl] * scale, zero)

    def grp(g, carry):
        one_group(pl.ds(g * 32, 16))
        one_group(pl.ds(g * 32 + 16, 16))
        return carry

    lax.fori_loop(0, NGRP // 2, grp, 0)
    pltpu.sync_copy(act_v, act_hbm.at[:, pl.ds(base, TOK_W)])


def _p3_first_kernel(act_ref, b_ref, o_ref):
    act = act_ref[...].astype(jnp.bfloat16)
    out = jax.lax.dot_general(
        act, b_ref[...], (((0,), (1,)), ((), ())),
        preferred_element_type=jnp.float32)
    o_ref[...] = out


def _p3_rest_kernel(o_in_ref, act_ref, b_ref, o_ref):
    del o_in_ref
    _p3_first_kernel(act_ref, b_ref, o_ref)


def _p3_call(act_t, b_bf, chunk, prev_out):
    base = chunk * (CTOK // BL)
    out_spec = pl.BlockSpec((BL, OUT_F), lambda i: (base + i, 0))
    out_shape = jax.ShapeDtypeStruct((NTOK, OUT_F), jnp.float32)
    act_spec = pl.BlockSpec((RDIM, BL), lambda i: (0, i))
    b_spec = pl.BlockSpec((OUT_F, RDIM), lambda i: (0, 0))
    params = pltpu.CompilerParams(dimension_semantics=("arbitrary",))
    if prev_out is None:
        return pl.pallas_call(
            _p3_first_kernel,
            grid=(CTOK // BL,),
            in_specs=[act_spec, b_spec],
            out_specs=out_spec,
            out_shape=out_shape,
            compiler_params=params,
        )(act_t, b_bf)
    return pl.pallas_call(
        _p3_rest_kernel,
        grid=(CTOK // BL,),
        in_specs=[
            pl.BlockSpec(memory_space=pl.ANY),
            act_spec,
            b_spec,
        ],
        out_specs=out_spec,
        out_shape=out_shape,
        input_output_aliases={0: 0},
        compiler_params=params,
    )(prev_out, act_t, b_bf)


@jax.jit
def kernel(x, A, B, d):
    d2 = d.reshape(1, RDIM)
    b_bf = B.astype(jnp.bfloat16)
    ys = [_p1_call(x, A, d2, c) for c in range(NCHUNK)]
    acts = [_sc_topk(yt, ybt) for (yt, ybt) in ys]
    out = None
    for c in range(NCHUNK):
        out = _p3_call(acts[c], b_bf, c, out)
    return out


# speedup vs baseline: 1.7348x; 1.4786x over previous
import jax
import jax.numpy as jnp
from jax.experimental import pallas as pl
from jax.experimental.pallas import tpu as pltpu

IN_F = 2048
OUT_F = 2048
RDIM = 32
KSEL = 8
SCALE = 64.0 / 32.0


def _routing_matmuls(x_blk, a, b, d):
    y = jax.lax.dot_general(
        x_blk, a, (((1,), (1,)), ((), ())),
        preferred_element_type=jnp.float32)
    yb = jnp.abs(y + d)

    keys = jnp.transpose(yb).view(jnp.int32)
    row = jax.lax.broadcasted_iota(jnp.int32, (RDIM, keys.shape[1]), 0)
    rank = jnp.zeros(keys.shape, jnp.int32)
    for j in range(RDIM):
        kj = jnp.zeros_like(keys) + keys[j:j + 1, :]
        gt = (kj > keys).astype(jnp.int32)
        ge = (kj >= keys).astype(jnp.int32)
        rank = rank + jnp.where(row > j, ge, gt)
    mask_t = jnp.where(rank < KSEL, jnp.float32(SCALE), jnp.float32(0.0))
    mask = jnp.transpose(mask_t)

    act = (y * mask).astype(jnp.bfloat16)
    return jax.lax.dot_general(
        act, b, (((1,), (1,)), ((), ())),
        preferred_element_type=jnp.float32)


def _fused_kernel(x_ref, a_ref, b_ref, d_ref, o_ref):
    a = a_ref[...]
    b = b_ref[...]
    d = d_ref[...]
    bl = x_ref.shape[0]
    h = bl // 8
    for c in range(8):
        o_ref[c * h:(c + 1) * h, :] = _routing_matmuls(
            x_ref[c * h:(c + 1) * h, :], a, b, d)


def _fused_call(x, A, b_bf, d2, bl):
    n_tokens = x.shape[0]
    grid = (n_tokens // bl,)
    return pl.pallas_call(
        _fused_kernel,
        grid=grid,
        in_specs=[
            pl.BlockSpec((bl, IN_F), lambda i: (i, 0)),
            pl.BlockSpec((RDIM, IN_F), lambda i: (0, 0)),
            pl.BlockSpec((OUT_F, RDIM), lambda i: (0, 0)),
            pl.BlockSpec((1, RDIM), lambda i: (0, 0)),
        ],
        out_specs=pl.BlockSpec((bl, OUT_F), lambda i: (i, 0)),
        out_shape=jax.ShapeDtypeStruct((n_tokens, OUT_F), jnp.float32),
        compiler_params=pltpu.CompilerParams(
            dimension_semantics=("parallel",)),
    )(x, A, b_bf, d2)


@jax.jit
def kernel(x, A, B, d):
    d2 = d.reshape(1, RDIM)
    b_bf = B.astype(jnp.bfloat16)
    return _fused_call(x, A, b_bf, d2, bl=1024)


# speedup vs baseline: 1.7618x; 1.0156x over previous
import jax
import jax.numpy as jnp
from jax.experimental import pallas as pl
from jax.experimental.pallas import tpu as pltpu

IN_F = 2048
OUT_F = 2048
RDIM = 32
KSEL = 8
SCALE = 64.0 / 32.0


def _routing_matmuls(x_blk, a, b, d):
    y = jax.lax.dot_general(
        x_blk, a, (((1,), (1,)), ((), ())),
        preferred_element_type=jnp.float32)
    yb = jnp.abs(y + d)

    keys = jnp.transpose(yb).view(jnp.int32)
    row = jax.lax.broadcasted_iota(jnp.int32, (RDIM, keys.shape[1]), 0)
    rank = jnp.zeros(keys.shape, jnp.int32)
    for j in range(RDIM):
        kj = keys[j:j + 1, :]
        gt = (kj > keys).astype(jnp.int32)
        ge = (kj >= keys).astype(jnp.int32)
        rank = rank + jnp.where(row > j, ge, gt)
    mask_t = jnp.where(rank < KSEL, jnp.float32(SCALE), jnp.float32(0.0))
    mask = jnp.transpose(mask_t)

    act = (y * mask).astype(jnp.bfloat16)
    return jax.lax.dot_general(
        act, b, (((1,), (1,)), ((), ())),
        preferred_element_type=jnp.float32)


def _fused_kernel(x_ref, a_ref, b_ref, d_ref, o_ref):
    a = a_ref[...]
    b = b_ref[...]
    d = d_ref[...]
    bl = x_ref.shape[0]
    h = bl // 8
    for c in range(8):
        o_ref[c * h:(c + 1) * h, :] = _routing_matmuls(
            x_ref[c * h:(c + 1) * h, :], a, b, d)


def _fused_call(x, A, b_bf, d2, bl):
    n_tokens = x.shape[0]
    grid = (n_tokens // bl,)
    return pl.pallas_call(
        _fused_kernel,
        grid=grid,
        in_specs=[
            pl.BlockSpec((bl, IN_F), lambda i: (i, 0)),
            pl.BlockSpec((RDIM, IN_F), lambda i: (0, 0)),
            pl.BlockSpec((OUT_F, RDIM), lambda i: (0, 0)),
            pl.BlockSpec((1, RDIM), lambda i: (0, 0)),
        ],
        out_specs=pl.BlockSpec((bl, OUT_F), lambda i: (i, 0)),
        out_shape=jax.ShapeDtypeStruct((n_tokens, OUT_F), jnp.float32),
        compiler_params=pltpu.CompilerParams(
            dimension_semantics=("parallel",)),
    )(x, A, b_bf, d2)


@jax.jit
def kernel(x, A, B, d):
    d2 = d.reshape(1, RDIM)
    b_bf = B.astype(jnp.bfloat16)
    return _fused_call(x, A, b_bf, d2, bl=1024)
